# Initial kernel scaffold; baseline (speedup 1.0000x reference)
import jax
import jax.numpy as jnp
from jax.experimental import pallas as pl


def kernel(combined_features, w_proj, b_proj, w_ffn1, b_ffn1, bn1_g, bn1_b, w_ffn2, b_ffn2, bn2_g, bn2_b, ln_g, ln_b):
    raise NotImplementedError("write your pallas kernel here")


# speedup vs baseline: 1.9375x; 1.9375x over previous
import jax
import jax.numpy as jnp
from jax.experimental import pallas as pl
from jax.experimental.pallas import tpu as pltpu


def _mm(a, b):
    return jax.lax.dot_general(a, b, (((1,), (0,)), ((), ())),
                               preferred_element_type=jnp.float32)


def _shift_w(a, dx):
    if dx == 0:
        return a
    rolled = jnp.roll(a, -dx, axis=-1)
    lane = jax.lax.broadcasted_iota(jnp.int32, (1, 1, a.shape[-1]), 2)
    edge = a.shape[-1] - 1 if dx > 0 else 0
    return jnp.where(lane == edge, 0.0, rolled)


def _gelu_exact(x):
    return 0.5 * x * (1.0 + jax.lax.erf(x * 0.7071067811865476))


def kernel(combined_features, w_proj, b_proj, w_ffn1, b_ffn1, bn1_g, bn1_b,
           w_ffn2, b_ffn2, bn2_g, bn2_b, ln_g, ln_b):
    B, N, C, H, W = combined_features.shape
    F = w_ffn1.shape[0]
    TH = 8
    T = H // TH

    wp = w_proj.reshape(w_proj.shape[0], N * C)
    bp = b_proj.reshape(-1, 1)
    w1 = (w_ffn1 * bn1_g[:, None, None, None]).transpose(2, 3, 0, 1)
    w1 = w1.reshape(9, F, C)
    b1 = (b_ffn1 * bn1_g + bn1_b).reshape(F, 1)
    w2 = (w_ffn2 * bn2_g[:, None, None, None]).transpose(2, 3, 0, 1)
    w2 = w2.reshape(9, C, F)
    b2 = (b_ffn2 * bn2_g + bn2_b).reshape(C, 1)
    lg = ln_g.reshape(C, 1)
    lb = ln_b.reshape(C, 1)

    def body(x_ref, wp_ref, bp_ref, w1_ref, b1_ref, w2_ref, b2_ref,
             lg_ref, lb_ref, out_ref, fprev, o1prev):
        i = pl.program_id(1)

        @pl.when(i == 0)
        def _init():
            fprev[...] = jnp.zeros_like(fprev)
            o1prev[...] = jnp.zeros_like(o1prev)

        x = x_ref[0]
        xc = x.reshape(N * C, TH * W)
        logits = _mm(wp_ref[...], xc) + bp_ref[...]
        m = jnp.max(logits, axis=0, keepdims=True)
        e = jnp.exp(logits - m)
        wn = e / jnp.sum(e, axis=0, keepdims=True)
        xr = x.reshape(N, C, TH * W)
        fused = (wn[0:1] * xr[0] + wn[1:2] * xr[1] + wn[2:3] * xr[2])
        fused = jnp.where(i < T, fused, 0.0)
        fused3 = fused.reshape(C, TH, W)

        halo_f = jnp.concatenate([fprev[:, TH - 2:, :], fused3], axis=1)
        sh_f = [_shift_w(halo_f, dx) for dx in (-1, 0, 1)]
        acc1 = jnp.zeros((F, TH * W), jnp.float32) + b1_ref[...]
        for k in range(9):
            dy, dx = divmod(k, 3)
            xsl = sh_f[dx][:, dy:dy + TH, :].reshape(C, TH * W)
            acc1 = acc1 + _mm(w1_ref[k], xsl)
        out1 = _gelu_exact(acc1).reshape(F, TH, W)
        ri = jax.lax.broadcasted_iota(jnp.int32, (1, TH, 1), 1) + i * TH - 1
        out1 = jnp.where((ri >= 0) & (ri < H), out1, 0.0)

        @pl.when(i > 0)
        def _emit():
            halo_g = jnp.concatenate([o1prev[...], out1[:, :2, :]], axis=1)
            sh_g = [_shift_w(halo_g, dx) for dx in (-1, 0, 1)]
            acc2 = jnp.zeros((C, TH * W), jnp.float32) + b2_ref[...]
            for k in range(9):
                dy, dx = divmod(k, 3)
                xsl = sh_g[dx][:, dy:dy + TH, :].reshape(F, TH * W)
                acc2 = acc2 + _mm(w2_ref[k], xsl)
            acc2 = acc2 + fprev[...].reshape(C, TH * W)
            mu = jnp.mean(acc2, axis=0, keepdims=True)
            cen = acc2 - mu
            var = jnp.mean(cen * cen, axis=0, keepdims=True)
            y = cen * jax.lax.rsqrt(var + 1e-5) * lg_ref[...] + lb_ref[...]
            out_ref[0] = y.reshape(C, TH, W)

        fprev[...] = fused3
        o1prev[...] = out1

    grid = (B, T + 1)
    out = pl.pallas_call(
        body,
        grid=grid,
        in_specs=[
            pl.BlockSpec((1, N, C, TH, W),
                         lambda b, i: (b, 0, 0, jnp.minimum(i, T - 1), 0)),
            pl.BlockSpec(wp.shape, lambda b, i: (0, 0)),
            pl.BlockSpec(bp.shape, lambda b, i: (0, 0)),
            pl.BlockSpec(w1.shape, lambda b, i: (0, 0, 0)),
            pl.BlockSpec(b1.shape, lambda b, i: (0, 0)),
            pl.BlockSpec(w2.shape, lambda b, i: (0, 0, 0)),
            pl.BlockSpec(b2.shape, lambda b, i: (0, 0)),
            pl.BlockSpec(lg.shape, lambda b, i: (0, 0)),
            pl.BlockSpec(lb.shape, lambda b, i: (0, 0)),
        ],
        out_specs=pl.BlockSpec((1, C, TH, W),
                               lambda b, i: (b, 0, jnp.maximum(i - 1, 0), 0)),
        out_shape=jax.ShapeDtypeStruct((B, C, H, W), jnp.float32),
        scratch_shapes=[
            pltpu.VMEM((C, TH, W), jnp.float32),
            pltpu.VMEM((F, TH, W), jnp.float32),
        ],
        compiler_params=pltpu.CompilerParams(
            dimension_semantics=("arbitrary", "arbitrary"),
            vmem_limit_bytes=100 * 1024 * 1024,
        ),
    )(combined_features, wp, bp, w1, b1, w2, b2, lg, lb)
    return out


# speedup vs baseline: 2.3602x; 1.2182x over previous
import jax
import jax.numpy as jnp
from jax.experimental import pallas as pl
from jax.experimental.pallas import tpu as pltpu


def _mm(a, b):
    return jax.lax.dot_general(a, b, (((1,), (0,)), ((), ())),
                               preferred_element_type=jnp.float32)


def _shift_w(a, dx):
    if dx == 0:
        return a
    rolled = jnp.roll(a, -dx, axis=-1)
    lane = jax.lax.broadcasted_iota(jnp.int32, (1, 1, a.shape[-1]), 2)
    edge = a.shape[-1] - 1 if dx > 0 else 0
    return jnp.where(lane == edge, 0.0, rolled)


def _gelu_exact(x):
    return 0.5 * x * (1.0 + jax.lax.erf(x * 0.7071067811865476))


def kernel(combined_features, w_proj, b_proj, w_ffn1, b_ffn1, bn1_g, bn1_b,
           w_ffn2, b_ffn2, bn2_g, bn2_b, ln_g, ln_b):
    B, N, C, H, W = combined_features.shape
    F = w_ffn1.shape[0]
    TH = 8
    T = H // TH

    wp = w_proj.reshape(w_proj.shape[0], N * C)
    bp = b_proj.reshape(-1, 1)
    w1 = (w_ffn1 * bn1_g[:, None, None, None]).transpose(2, 3, 0, 1)
    w1 = w1.reshape(9, F, C).astype(jnp.bfloat16)
    b1 = (b_ffn1 * bn1_g + bn1_b).reshape(F, 1)
    w2 = (w_ffn2 * bn2_g[:, None, None, None]).transpose(2, 3, 0, 1)
    w2 = w2.reshape(9, C, F).astype(jnp.bfloat16)
    b2 = (b_ffn2 * bn2_g + bn2_b).reshape(C, 1)
    lg = ln_g.reshape(C, 1)
    lb = ln_b.reshape(C, 1)

    def body(x_ref, wp_ref, bp_ref, w1_ref, b1_ref, w2_ref, b2_ref,
             lg_ref, lb_ref, out_ref, fprev, o1prev):
        i = pl.program_id(1)

        @pl.when(i == 0)
        def _init():
            fprev[...] = jnp.zeros_like(fprev)
            o1prev[...] = jnp.zeros_like(o1prev)

        x = x_ref[0]
        xc = x.reshape(N * C, TH * W)
        logits = _mm(wp_ref[...], xc) + bp_ref[...]
        m = jnp.max(logits, axis=0, keepdims=True)
        e = jnp.exp(logits - m)
        wn = e / jnp.sum(e, axis=0, keepdims=True)
        xr = x.reshape(N, C, TH * W)
        fused = (wn[0:1] * xr[0] + wn[1:2] * xr[1] + wn[2:3] * xr[2])
        fused = jnp.where(i < T, fused, 0.0)
        fused3 = fused.reshape(C, TH, W)

        halo_f = jnp.concatenate([fprev[:, TH - 2:, :], fused3], axis=1)
        halo_f = halo_f.astype(jnp.bfloat16)
        sh_f = [_shift_w(halo_f, dx) for dx in (-1, 0, 1)]
        acc1 = jnp.zeros((F, TH * W), jnp.float32) + b1_ref[...]
        for k in range(9):
            dy, dx = divmod(k, 3)
            xsl = sh_f[dx][:, dy:dy + TH, :].reshape(C, TH * W)
            acc1 = acc1 + _mm(w1_ref[k], xsl)
        out1 = _gelu_exact(acc1).reshape(F, TH, W)
        ri = jax.lax.broadcasted_iota(jnp.int32, (1, TH, 1), 1) + i * TH - 1
        out1 = jnp.where((ri >= 0) & (ri < H), out1, 0.0)

        @pl.when(i > 0)
        def _emit():
            halo_g = jnp.concatenate([o1prev[...], out1[:, :2, :]], axis=1)
            halo_g = halo_g.astype(jnp.bfloat16)
            sh_g = [_shift_w(halo_g, dx) for dx in (-1, 0, 1)]
            acc2 = jnp.zeros((C, TH * W), jnp.float32) + b2_ref[...]
            for k in range(9):
                dy, dx = divmod(k, 3)
                xsl = sh_g[dx][:, dy:dy + TH, :].reshape(F, TH * W)
                acc2 = acc2 + _mm(w2_ref[k], xsl)
            acc2 = acc2 + fprev[...].reshape(C, TH * W)
            mu = jnp.mean(acc2, axis=0, keepdims=True)
            cen = acc2 - mu
            var = jnp.mean(cen * cen, axis=0, keepdims=True)
            y = cen * jax.lax.rsqrt(var + 1e-5) * lg_ref[...] + lb_ref[...]
            out_ref[0] = y.reshape(C, TH, W)

        fprev[...] = fused3
        o1prev[...] = out1

    grid = (B, T + 1)
    out = pl.pallas_call(
        body,
        grid=grid,
        in_specs=[
            pl.BlockSpec((1, N, C, TH, W),
                         lambda b, i: (b, 0, 0, jnp.minimum(i, T - 1), 0)),
            pl.BlockSpec(wp.shape, lambda b, i: (0, 0)),
            pl.BlockSpec(bp.shape, lambda b, i: (0, 0)),
            pl.BlockSpec(w1.shape, lambda b, i: (0, 0, 0)),
            pl.BlockSpec(b1.shape, lambda b, i: (0, 0)),
            pl.BlockSpec(w2.shape, lambda b, i: (0, 0, 0)),
            pl.BlockSpec(b2.shape, lambda b, i: (0, 0)),
            pl.BlockSpec(lg.shape, lambda b, i: (0, 0)),
            pl.BlockSpec(lb.shape, lambda b, i: (0, 0)),
        ],
        out_specs=pl.BlockSpec((1, C, TH, W),
                               lambda b, i: (b, 0, jnp.maximum(i - 1, 0), 0)),
        out_shape=jax.ShapeDtypeStruct((B, C, H, W), jnp.float32),
        scratch_shapes=[
            pltpu.VMEM((C, TH, W), jnp.float32),
            pltpu.VMEM((F, TH, W), jnp.float32),
        ],
        compiler_params=pltpu.CompilerParams(
            dimension_semantics=("arbitrary", "arbitrary"),
            vmem_limit_bytes=100 * 1024 * 1024,
        ),
    )(combined_features, wp, bp, w1, b1, w2, b2, lg, lb)
    return out
